# baseline (speedup 1.0000x reference)
import dataclasses
import functools

import jax
import jax.numpy as jnp
from jax import lax
from jax.experimental import pallas as pl
from jax.experimental.pallas import tpu as pltpu
from jax.experimental.pallas import tpu_sc as plsc

N = 10000
N2 = 10016
E = 320000
D = 128
DW = 16
C = 32
NPROC = 320
NWORK = 32
NPAIR = NPROC // 2
SROWS = 40
HALF = SROWS // 2
HPAIRS = HALF // 2
NWIN = NPROC // HALF
PAD_ROWS = NWORK * NPROC + HALF

_SC_COMPILER_PARAMS = pltpu.CompilerParams(use_tc_tiling_on_sc=False)
if "needs_layout_passes" in pltpu.CompilerParams.__dataclass_fields__:
    _SC_COMPILER_PARAMS = dataclasses.replace(_SC_COMPILER_PARAMS, needs_layout_passes=False)

ROWS_PER_TILE = 624
SLAB = 24
DSLAB = 104
TAIL = N - 16 * ROWS_PER_TILE


def _scalar_table_body(fa_ref, fs_ref, fp_ref, lap_ref, rap_ref, lsp_ref, rsp_ref, out_ref):
    fa = fa_ref[...]
    fs = fs_ref[...]
    fp = fp_ref[...]
    el_ap = jnp.sum(fa * lap_ref[...][0][None, :], axis=1)
    er_ap = jnp.sum(fp * rap_ref[...][0][None, :], axis=1)
    el_sp = jnp.sum(fs * lsp_ref[...][0][None, :], axis=1)
    er_sp = jnp.sum(fp * rsp_ref[...][0][None, :], axis=1)
    out_ref[...] = jnp.stack([el_ap, er_ap, el_sp, er_sp], axis=0)


def _scalar_table(feat_author, feat_subject, feat_paper, attn_l_ap, attn_r_ap, attn_l_sp, attn_r_sp):
    return pl.pallas_call(
        _scalar_table_body,
        out_shape=jax.ShapeDtypeStruct((4, N), jnp.float32),
    )(feat_author, feat_subject, feat_paper, attn_l_ap, attn_r_ap, attn_l_sp, attn_r_sp)


def _sc_gat(scal, sd_ap, sd_sp, feat_a, feat_s):
    mesh = plsc.VectorSubcoreMesh(core_axis_name="c", subcore_axis_name="s")

    @functools.partial(
        pl.kernel,
        out_type=[
            jax.ShapeDtypeStruct((2, N, D), jnp.float32),
            jax.ShapeDtypeStruct((2, N, DW), jnp.float32),
            jax.ShapeDtypeStruct((2, N, D), jnp.float32),
            jax.ShapeDtypeStruct((2, N, DW), jnp.float32),
        ],
        mesh=mesh,
        scratch_types=[
            pltpu.VMEM((N2,), jnp.float32),
            pltpu.VMEM((N2,), jnp.float32),
            pltpu.VMEM((SROWS, C), jnp.int32),
            pltpu.VMEM((4, C), jnp.int32),
            pltpu.VMEM((4, C), jnp.int32),
            pltpu.VMEM((C, D), jnp.float32),
            pltpu.VMEM((C, D), jnp.float32),
            pltpu.VMEM((C, D), jnp.float32),
            pltpu.VMEM((C, D), jnp.float32),
            pltpu.VMEM((C, DW), jnp.float32),
            pltpu.VMEM((C, DW), jnp.float32),
            pltpu.VMEM((C,), jnp.float32),
            pltpu.VMEM_SHARED((N2, D), jnp.float32),
            pltpu.VMEM_SHARED((N2, DW), jnp.float32),
            pltpu.SemaphoreType.DMA,
            pltpu.SemaphoreType.DMA,
            pltpu.SemaphoreType.DMA,
            pltpu.SemaphoreType.DMA,
            pltpu.SemaphoreType.DMA,
            pltpu.SemaphoreType.DMA,
            pltpu.SemaphoreType.DMA,
        ],
        compiler_params=_SC_COMPILER_PARAMS,
    )
    def kern(scal_hbm, sd_ap_hbm, sd_sp_hbm, feat_a_hbm, feat_s_hbm,
             out_ap_hbm, den_ap_hbm, out_sp_hbm, den_sp_hbm,
             el_t, er_t, stash, sidx, didx, g0, g1, s0, s1, pr0, pr1, p_col,
             acc, accd, gs0, gs1, ss0, ss1, ps0, ps1, rsem):
        cid = lax.axis_index("c")
        sid = lax.axis_index("s")
        wid = sid * 2 + cid
        base = wid * NPROC

        col_iota = lax.iota(jnp.int32, 16)
        denom_mask = jnp.where(col_iota == 0, 1.0, 0.0).astype(jnp.float32)
        zz = jnp.zeros((16,), jnp.float32)

        def zero_stage0():
            @pl.loop(0, C)
            def _(r):
                for g in range(D // 16):
                    s0[r, pl.ds(16 * g, 16)] = zz
                pr0[r, :] = zz

        def zero_acc():
            for k in range(ROWS_PER_TILE // SLAB):
                r0 = sid * ROWS_PER_TILE + k * SLAB
                pltpu.sync_copy(s0.at[pl.ds(0, SLAB), :], acc.at[pl.ds(r0, SLAB), :])
                pltpu.sync_copy(pr0.at[pl.ds(0, SLAB), :], accd.at[pl.ds(r0, SLAB), :])

            @pl.when(sid == 15)
            def _():
                r0 = 16 * ROWS_PER_TILE
                pltpu.sync_copy(s0.at[pl.ds(0, TAIL), :], acc.at[pl.ds(r0, TAIL), :])
                pltpu.sync_copy(pr0.at[pl.ds(0, TAIL), :], accd.at[pl.ds(r0, TAIL), :])

        def drain(out_hbm, den_hbm):
            for k in range(ROWS_PER_TILE // DSLAB):
                r0 = sid * ROWS_PER_TILE + k * DSLAB
                pltpu.sync_copy(acc.at[pl.ds(r0, DSLAB), :], out_hbm.at[cid].at[pl.ds(r0, DSLAB), :])
                pltpu.sync_copy(accd.at[pl.ds(r0, DSLAB), :], den_hbm.at[cid].at[pl.ds(r0, DSLAB), :])

            @pl.when(sid == 15)
            def _():
                r0 = 16 * ROWS_PER_TILE
                pltpu.sync_copy(acc.at[pl.ds(r0, TAIL), :], out_hbm.at[cid].at[pl.ds(r0, TAIL), :])
                pltpu.sync_copy(accd.at[pl.ds(r0, TAIL), :], den_hbm.at[cid].at[pl.ds(r0, TAIL), :])

        def run_relation(sd_hbm, feat_hbm, el_row, er_row, out_hbm, den_hbm):
            pltpu.sync_copy(scal_hbm.at[el_row], el_t)
            pltpu.sync_copy(scal_hbm.at[er_row], er_t)

            def store_idx(row, slot):
                for g in range(C // 16):
                    sd = stash[row, pl.ds(16 * g, 16)]
                    sidx[slot, pl.ds(16 * g, 16)] = sd & 0xFFFF
                    didx[slot, pl.ds(16 * g, 16)] = sd >> 16

            def gather_start(slot, gref, gsem):
                pltpu.async_copy(feat_hbm.at[sidx.at[slot]], gref, gsem)

            def gather_wait(slot, gref, gsem):
                pltpu.make_async_copy(feat_hbm.at[sidx.at[slot]], gref, gsem).wait()

            def scatter_start(slot, sref, pref, ssem, psem):
                pltpu.async_copy(sref, acc.at[didx.at[slot]], ssem, add=True)
                pltpu.async_copy(pref, accd.at[didx.at[slot]], psem, add=True)

            def scatter_wait(slot, sref, pref, ssem, psem):
                pltpu.make_async_copy(sref, acc.at[didx.at[slot]], ssem).wait()
                pltpu.make_async_copy(pref, accd.at[didx.at[slot]], psem).wait()

            def refill_start(win):
                h0 = (win % 2) * HALF
                pltpu.async_copy(sd_hbm.at[pl.ds(base + win * HALF, HALF), :],
                                 stash.at[pl.ds(h0, HALF), :], rsem)

            def refill_wait(win):
                h0 = (win % 2) * HALF
                pltpu.make_async_copy(sd_hbm.at[pl.ds(base + win * HALF, HALF), :],
                                      stash.at[pl.ds(h0, HALF), :], rsem).wait()

            def compute(row, gref, sref, pref):
                for g in range(C // 16):
                    sd = stash[row, pl.ds(16 * g, 16)]
                    e = (plsc.load_gather(el_t, [sd & 0xFFFF])
                         + plsc.load_gather(er_t, [sd >> 16]))
                    e = jnp.where(e >= 0.0, e, 0.01 * e)
                    p_col[pl.ds(16 * g, 16)] = jnp.exp(e)

                @pl.loop(0, C, unroll=4)
                def _(r):
                    psp = plsc.load_gather(p_col, [jnp.full((16,), r, jnp.int32)])
                    for g in range(D // 16):
                        sref[r, pl.ds(16 * g, 16)] = gref[r, pl.ds(16 * g, 16)] * psp
                    pref[r, :] = psp * denom_mask

            pltpu.sync_copy(sd_hbm.at[pl.ds(base, HALF), :], stash.at[pl.ds(0, HALF), :])
            refill_start(1)
            store_idx(0, 0)
            gather_start(0, g0, gs0)

            @pl.loop(0, NPAIR)
            def _(t):
                c0 = 2 * t
                row0 = lax.rem(c0, SROWS)
                slot0 = lax.rem(c0, 4)

                store_idx(row0 + 1, slot0 + 1)
                gather_start(slot0 + 1, g1, gs1)
                gather_wait(slot0, g0, gs0)

                compute(row0, g0, s0, pr0)

                @pl.when(t > 0)
                def _():
                    scatter_wait(lax.rem(c0 + 3, 4), s1, pr1, ss1, ps1)

                scatter_start(slot0, s0, pr0, ss0, ps0)

                tm = lax.rem(t, HPAIRS)

                @pl.when((tm == HPAIRS - 1) & (t < NPAIR - 1))
                def _():
                    refill_wait(lax.div(t, HPAIRS) + 1)

                @pl.when(t < NPAIR - 1)
                def _():
                    store_idx(lax.rem(c0 + 2, SROWS), lax.rem(c0 + 2, 4))
                    gather_start(lax.rem(c0 + 2, 4), g0, gs0)

                gather_wait(slot0 + 1, g1, gs1)
                compute(row0 + 1, g1, s1, pr1)
                scatter_wait(slot0, s0, pr0, ss0, ps0)
                scatter_start(slot0 + 1, s1, pr1, ss1, ps1)

                @pl.when((tm == HPAIRS - 1) & (t < NPAIR - 2 * HPAIRS))
                def _():
                    refill_start(lax.div(t, HPAIRS) + 2)

            scatter_wait((NPROC - 1) % 4, s1, pr1, ss1, ps1)

        zero_stage0()
        zero_acc()
        plsc.subcore_barrier()
        run_relation(sd_ap_hbm, feat_a_hbm, 0, 1, out_ap_hbm, den_ap_hbm)
        plsc.subcore_barrier()
        drain(out_ap_hbm, den_ap_hbm)
        zero_stage0()
        zero_acc()
        plsc.subcore_barrier()
        run_relation(sd_sp_hbm, feat_s_hbm, 2, 3, out_sp_hbm, den_sp_hbm)
        plsc.subcore_barrier()
        drain(out_sp_hbm, den_sp_hbm)

    return kern(scal, sd_ap, sd_sp, feat_a, feat_s)


ROWS_TC = 1000
GRID_TC = N // ROWS_TC


def _post_body(ap_ref, dap_ref, sp_ref, dsp_ref, fcw_ref, fcb_ref,
               h_ap_ref, h_sp_ref, tsum_ref):
    step = pl.program_id(0)

    @pl.when(step == 0)
    def _():
        tsum_ref[...] = jnp.zeros_like(tsum_ref)

    fcw = fcw_ref[...]
    fcb = fcb_ref[...]
    for m, (part_ref, den_ref, h_ref) in enumerate(
            ((ap_ref, dap_ref, h_ap_ref), (sp_ref, dsp_ref, h_sp_ref))):
        num = part_ref[...][0] + part_ref[...][1]
        denf = den_ref[...][0] + den_ref[...][1]
        den = denf[:, 0:1]
        h = jnp.where(den > 0.0, num / jnp.where(den > 0.0, den, 1.0), 0.0)
        h = jnp.where(h > 0.0, h, jnp.exp(h) - 1.0)
        h_ref[...] = h
        t = jnp.tanh(
            jax.lax.dot_general(h, fcw, (((1,), (1,)), ((), ())),
                                preferred_element_type=jnp.float32) + fcb[None, :])
        tsum_ref[pl.ds(m, 1), :] += jnp.sum(t, axis=0, keepdims=True)


def _post(out_ap, den_ap, out_sp, den_sp, fc_w, fc_b):
    return pl.pallas_call(
        _post_body,
        grid=(GRID_TC,),
        in_specs=[
            pl.BlockSpec((2, ROWS_TC, D), lambda i: (0, i, 0)),
            pl.BlockSpec((2, ROWS_TC, DW), lambda i: (0, i, 0)),
            pl.BlockSpec((2, ROWS_TC, D), lambda i: (0, i, 0)),
            pl.BlockSpec((2, ROWS_TC, DW), lambda i: (0, i, 0)),
            pl.BlockSpec((D, D), lambda i: (0, 0)),
            pl.BlockSpec((D,), lambda i: (0,)),
        ],
        out_specs=[
            pl.BlockSpec((ROWS_TC, D), lambda i: (i, 0)),
            pl.BlockSpec((ROWS_TC, D), lambda i: (i, 0)),
            pl.BlockSpec((2, D), lambda i: (0, 0)),
        ],
        out_shape=[
            jax.ShapeDtypeStruct((N, D), jnp.float32),
            jax.ShapeDtypeStruct((N, D), jnp.float32),
            jax.ShapeDtypeStruct((2, D), jnp.float32),
        ],
    )(out_ap, den_ap, out_sp, den_sp, fc_w, fc_b)


def _combine_body(h_ap_ref, h_sp_ref, tsum_ref, sem_ref, out_ref):
    tmean = tsum_ref[...] * (1.0 / N)
    a = sem_ref[...][0]
    w0 = jnp.sum(tmean[0] * a)
    w1 = jnp.sum(tmean[1] * a)
    m = jnp.maximum(w0, w1)
    b0 = jnp.exp(w0 - m)
    b1 = jnp.exp(w1 - m)
    s = b0 + b1
    out_ref[...] = (b0 * h_ap_ref[...] + b1 * h_sp_ref[...]) / s


def _combine(h_ap, h_sp, tsum, attn_sem):
    return pl.pallas_call(
        _combine_body,
        grid=(GRID_TC,),
        in_specs=[
            pl.BlockSpec((ROWS_TC, D), lambda i: (i, 0)),
            pl.BlockSpec((ROWS_TC, D), lambda i: (i, 0)),
            pl.BlockSpec((2, D), lambda i: (0, 0)),
            pl.BlockSpec((1, D), lambda i: (0, 0)),
        ],
        out_specs=pl.BlockSpec((ROWS_TC, D), lambda i: (i, 0)),
        out_shape=jax.ShapeDtypeStruct((N, D), jnp.float32),
    )(h_ap, h_sp, tsum, attn_sem)


def _pack_edges(edge_index):
    sd = edge_index[0] + (edge_index[1] << 16)
    sd = sd.reshape(E // C, C)
    pad_val = jnp.int32(N + (N << 16))
    return jnp.pad(sd, ((0, PAD_ROWS - E // C), (0, 0)), constant_values=pad_val)


def kernel(feat_author, feat_subject, feat_paper, edge_index_ap, edge_index_sp,
           attn_l_ap, attn_r_ap, attn_l_sp, attn_r_sp, fc_w, fc_b, attn_sem):
    scal = _scalar_table(feat_author, feat_subject, feat_paper,
                         attn_l_ap, attn_r_ap, attn_l_sp, attn_r_sp)
    scal = jnp.pad(scal, ((0, 0), (0, N2 - N)))
    sd_ap = _pack_edges(edge_index_ap)
    sd_sp = _pack_edges(edge_index_sp)
    feat_a = jnp.pad(feat_author, ((0, N2 - N), (0, 0)))
    feat_s = jnp.pad(feat_subject, ((0, N2 - N), (0, 0)))
    out_ap, den_ap, out_sp, den_sp = _sc_gat(scal, sd_ap, sd_sp, feat_a, feat_s)
    h_ap, h_sp, tsum = _post(out_ap, den_ap, out_sp, den_sp, fc_w, fc_b)
    return _combine(h_ap, h_sp, tsum, attn_sem)
